# baseline (speedup 1.0000x reference)
import functools

import jax
import jax.numpy as jnp
from jax import lax
from jax.experimental import pallas as pl
from jax.experimental.pallas import tpu as pltpu
from jax.experimental.pallas import tpu_sc as plsc
from jax._src.pallas import mpmd as _mpmd

N_NODES = 100000
D_MEM = 128
D_MSG = 256
B = 16384

NW = 32
BPW = B // NW
CH = 128
RANGE = 3136
LAST_RANGE = N_NODES - 31 * RANGE
TBL_V = RANGE // 16
KMAX = (RANGE + CH - 1) // CH
WFLAT = KMAX * CH + 16

_mesh = plsc.VectorSubcoreMesh(core_axis_name="c", subcore_axis_name="s")


def _wid():
    return lax.axis_index("s") * 2 + lax.axis_index("c")


def _gather_body(mem_hbm, idx_hbm, h_hbm, idx2, rows_v, sem):
    base = _wid() * BPW
    for k in range(BPW // CH):
        pltpu.sync_copy(idx_hbm.at[pl.ds(base + k * CH, CH)], idx2.at[k])
    cps = [
        pltpu.async_copy(mem_hbm.at[idx2.at[k]], rows_v.at[pl.ds(k * CH, CH)], sem)
        for k in range(BPW // CH)
    ]
    for cp in cps:
        cp.wait()
    pltpu.sync_copy(rows_v, h_hbm.at[pl.ds(base, BPW)])


_gather_call = pl.kernel(
    _gather_body,
    out_type=jax.ShapeDtypeStruct((B, D_MEM), jnp.float32),
    mesh=_mesh,
    scratch_types=[
        pltpu.VMEM((BPW // CH, CH), jnp.int32),
        pltpu.VMEM((BPW, D_MEM), jnp.float32),
        pltpu.SemaphoreType.DMA,
    ],
    name="sc_gather_rows",
)


BCOPY = 1000
NCOPY = N_NODES // BCOPY
BR = 512


def _gru_body(mem_ref, msgs_ref, h_ref, wih_ref, whh_ref, bih_ref, bhh_ref,
              out_mem_ref, upd_ref):
    i = pl.program_id(0)

    @pl.when(i < NCOPY)
    def _():
        out_mem_ref[...] = mem_ref[...]

    @pl.when(i >= NCOPY)
    def _():
        h = h_ref[...]
        gi = lax.dot_general(
            msgs_ref[...], wih_ref[...], (((1,), (1,)), ((), ())),
            preferred_element_type=jnp.float32,
        ) + bih_ref[...]
        gh = lax.dot_general(
            h, whh_ref[...], (((1,), (1,)), ((), ())),
            preferred_element_type=jnp.float32,
        ) + bhh_ref[...]
        r = jax.nn.sigmoid(gi[:, :D_MEM] + gh[:, :D_MEM])
        z = jax.nn.sigmoid(gi[:, D_MEM:2 * D_MEM] + gh[:, D_MEM:2 * D_MEM])
        n = jnp.tanh(gi[:, 2 * D_MEM:] + r * gh[:, 2 * D_MEM:])
        upd_ref[...] = (1.0 - z) * n + z * h


def _gru_call(memory, msgs, h, W_ih, W_hh, b_ih2, b_hh2):
    return pl.pallas_call(
        _gru_body,
        out_shape=(
            jax.ShapeDtypeStruct((N_NODES, D_MEM), jnp.float32),
            jax.ShapeDtypeStruct((B, D_MEM), jnp.float32),
        ),
        grid=(NCOPY + B // BR,),
        in_specs=[
            pl.BlockSpec((BCOPY, D_MEM), lambda i: (jnp.minimum(i, NCOPY - 1), 0)),
            pl.BlockSpec((BR, D_MSG), lambda i: (jnp.maximum(i - NCOPY, 0), 0)),
            pl.BlockSpec((BR, D_MEM), lambda i: (jnp.maximum(i - NCOPY, 0), 0)),
            pl.BlockSpec((3 * D_MEM, D_MSG), lambda i: (0, 0)),
            pl.BlockSpec((3 * D_MEM, D_MEM), lambda i: (0, 0)),
            pl.BlockSpec((1, 3 * D_MEM), lambda i: (0, 0)),
            pl.BlockSpec((1, 3 * D_MEM), lambda i: (0, 0)),
        ],
        out_specs=(
            pl.BlockSpec((BCOPY, D_MEM), lambda i: (jnp.minimum(i, NCOPY - 1), 0)),
            pl.BlockSpec((BR, D_MEM), lambda i: (jnp.maximum(i - NCOPY, 0), 0)),
        ),
        name="tc_copy_gru",
    )(memory, msgs, h, W_ih, W_hh, b_ih2, b_hh2)


def _scatter_body(mem_in, lu_in, idx_hbm, ts_hbm, upd_hbm, mem_out, lu_out,
                  ids_v, ts_v, cand, table, win_pos, win_id,
                  pos2d, id2d, lu_v, rows_v, gsem, ssem):
    wid = _wid()
    lo = wid * RANGE
    iota = lax.iota(jnp.int32, 16)

    pltpu.sync_copy(idx_hbm, ids_v)
    pltpu.sync_copy(ts_hbm, ts_v)
    is_last = wid == NW - 1
    hi = jnp.where(is_last, jnp.int32(N_NODES), lo + RANGE)

    def scan_step(j, cnt):
        v_id = ids_v[pl.ds(j * 16, 16)]
        m = (v_id >= lo) & (v_id < hi)
        packed = ((j * 16 + iota) << 17) | v_id
        pref = plsc.cumsum(m.astype(jnp.int32))
        plsc.store_scatter(cand, [cnt + pref - 1], packed, mask=m)
        return cnt + pref[15]

    cnt = lax.fori_loop(0, B // 16, scan_step, jnp.int32(0))

    def init_step(j, _):
        table[pl.ds(j * 16, 16)] = jnp.full((16,), -1, jnp.int32)
        return 0

    lax.fori_loop(0, TBL_V, init_step, 0)

    lane0 = iota == 0

    def dedup_step(t, _):
        vals = cand[pl.ds(t * 16, 16)]
        for k in range(16):
            val = vals[k]
            d = (val & 0x1FFFF) - lo
            pos = val >> 17
            mk = lane0 & (t * 16 + k < cnt)
            plsc.store_scatter(
                table,
                [jnp.full((16,), d, jnp.int32)],
                jnp.full((16,), pos, jnp.int32),
                mask=mk,
            )
        return 0

    lax.fori_loop(0, (cnt + 15) // 16, dedup_step, 0)

    @pl.when(~is_last)
    def _():
        pltpu.sync_copy(lu_in.at[pl.ds(lo, RANGE)], lu_v)

    @pl.when(is_last)
    def _():
        pltpu.sync_copy(lu_in.at[pl.ds(lo, LAST_RANGE)], lu_v.at[pl.ds(0, LAST_RANGE)])

    def collect_step(j, wcnt):
        v_pos = table[pl.ds(j * 16, 16)]
        m = v_pos >= 0
        v_id = lo + j * 16 + iota
        pref = plsc.cumsum(m.astype(jnp.int32))
        offs = wcnt + pref - 1
        plsc.store_scatter(win_pos, [offs], v_pos, mask=m)
        plsc.store_scatter(win_id, [offs], v_id, mask=m)
        g = plsc.load_gather(ts_v, [jnp.maximum(v_pos, 0)])
        cur = lu_v[pl.ds(j * 16, 16)]
        lu_v[pl.ds(j * 16, 16)] = jnp.where(m, g, cur)
        return wcnt + pref[15]

    wcnt = lax.fori_loop(0, TBL_V, collect_step, jnp.int32(0))

    @pl.when(~is_last)
    def _():
        pltpu.sync_copy(lu_v, lu_out.at[pl.ds(lo, RANGE)])

    @pl.when(is_last)
    def _():
        pltpu.sync_copy(lu_v.at[pl.ds(0, LAST_RANGE)], lu_out.at[pl.ds(lo, LAST_RANGE)])

    pos0 = win_pos[pl.ds(0, 16)][0]
    id0 = win_id[pl.ds(0, 16)][0]

    def xfer_step(j, _):
        valid = (j * 16 + iota) < wcnt
        vp = jnp.where(valid, win_pos[pl.ds(j * 16, 16)], pos0)
        vi = jnp.where(valid, win_id[pl.ds(j * 16, 16)], id0)
        c = j // (CH // 16)
        k = j % (CH // 16)
        pos2d[c, pl.ds(k * 16, 16)] = vp
        id2d[c, pl.ds(k * 16, 16)] = vi
        return 0

    lax.fori_loop(0, KMAX * (CH // 16), xfer_step, 0)

    nch = (wcnt + CH - 1) // CH

    def chunk_step(c, _):
        pltpu.async_copy(upd_hbm.at[pos2d.at[c]], rows_v, gsem).wait()
        pltpu.async_copy(rows_v, mem_out.at[id2d.at[c]], ssem).wait()
        return 0

    lax.fori_loop(0, nch, chunk_step, 0)


def _make_scatter():
    return _mpmd._mpmd_map(
        [(_mesh, _scatter_body)],
        (
            jax.ShapeDtypeStruct((N_NODES, D_MEM), jnp.float32),
            jax.ShapeDtypeStruct((N_NODES,), jnp.float32),
        ),
        input_output_aliases={0: 0, 1: 1},
        scratch_types=[
            pltpu.VMEM((B,), jnp.int32),
            pltpu.VMEM((B,), jnp.float32),
            pltpu.VMEM((B + 16,), jnp.int32),
            pltpu.VMEM((RANGE,), jnp.int32),
            pltpu.VMEM((WFLAT,), jnp.int32),
            pltpu.VMEM((WFLAT,), jnp.int32),
            pltpu.VMEM((KMAX, CH), jnp.int32),
            pltpu.VMEM((KMAX, CH), jnp.int32),
            pltpu.VMEM((RANGE,), jnp.float32),
            pltpu.VMEM((CH, D_MEM), jnp.float32),
            pltpu.SemaphoreType.DMA,
            pltpu.SemaphoreType.DMA,
        ],
        name="sc_scatter_rows",
        compiler_params=pltpu.CompilerParams(needs_layout_passes=False),
    )


_scatter = _make_scatter()


def kernel(memory, last_update, unique_node_ids, unique_messages, timestamps,
           W_ih, W_hh, b_ih, b_hh):
    ids = unique_node_ids.astype(jnp.int32)
    h = _gather_call(memory, ids)
    mem_copy, updated = _gru_call(
        memory, unique_messages, h, W_ih, W_hh,
        b_ih.reshape(1, 3 * D_MEM), b_hh.reshape(1, 3 * D_MEM),
    )
    new_mem, new_lu = _scatter(mem_copy, last_update, ids, timestamps, updated)
    return (new_mem, new_lu)


# speedup vs baseline: 1.5390x; 1.5390x over previous
import functools

import jax
import jax.numpy as jnp
from jax import lax
from jax.experimental import pallas as pl
from jax.experimental.pallas import tpu as pltpu
from jax.experimental.pallas import tpu_sc as plsc
from jax._src.pallas import mpmd as _mpmd

N_NODES = 100000
D_MEM = 128
D_MSG = 256
B = 16384

NW = 32
BPW = B // NW
CH = 128
RANGE = 3136
LAST_RANGE = N_NODES - 31 * RANGE
TBL_V = RANGE // 16
KMAX = (RANGE + CH - 1) // CH
WFLAT = KMAX * CH + 16
KPAD = 32

_mesh = plsc.VectorSubcoreMesh(core_axis_name="c", subcore_axis_name="s")


def _wid():
    return lax.axis_index("s") * 2 + lax.axis_index("c")


def _gather_body(mem_hbm, idx_hbm, h_hbm, idx2, rows_v, sem):
    base = _wid() * BPW
    for k in range(BPW // CH):
        pltpu.sync_copy(idx_hbm.at[pl.ds(base + k * CH, CH)], idx2.at[k])
    cps = [
        pltpu.async_copy(mem_hbm.at[idx2.at[k]], rows_v.at[pl.ds(k * CH, CH)], sem)
        for k in range(BPW // CH)
    ]
    for cp in cps:
        cp.wait()
    pltpu.sync_copy(rows_v, h_hbm.at[pl.ds(base, BPW)])


_gather_call = pl.kernel(
    _gather_body,
    out_type=jax.ShapeDtypeStruct((B, D_MEM), jnp.float32),
    mesh=_mesh,
    scratch_types=[
        pltpu.VMEM((BPW // CH, CH), jnp.int32),
        pltpu.VMEM((BPW, D_MEM), jnp.float32),
        pltpu.SemaphoreType.DMA,
    ],
    name="sc_gather_rows",
)


def _gru_body(msgs_ref, h_ref, wih_ref, whh_ref, bih_ref, bhh_ref, upd_ref):
    h = h_ref[...]
    gi = lax.dot_general(
        msgs_ref[...], wih_ref[...], (((1,), (1,)), ((), ())),
        preferred_element_type=jnp.float32,
    ) + bih_ref[...]
    gh = lax.dot_general(
        h, whh_ref[...], (((1,), (1,)), ((), ())),
        preferred_element_type=jnp.float32,
    ) + bhh_ref[...]
    r = jax.nn.sigmoid(gi[:, :D_MEM] + gh[:, :D_MEM])
    z = jax.nn.sigmoid(gi[:, D_MEM:2 * D_MEM] + gh[:, D_MEM:2 * D_MEM])
    n = jnp.tanh(gi[:, 2 * D_MEM:] + r * gh[:, 2 * D_MEM:])
    upd_ref[...] = (1.0 - z) * n + z * h


def _gru_call(msgs, h, W_ih, W_hh, b_ih2, b_hh2):
    BR = 512
    return pl.pallas_call(
        _gru_body,
        out_shape=jax.ShapeDtypeStruct((B, D_MEM), jnp.float32),
        grid=(B // BR,),
        in_specs=[
            pl.BlockSpec((BR, D_MSG), lambda i: (i, 0)),
            pl.BlockSpec((BR, D_MEM), lambda i: (i, 0)),
            pl.BlockSpec((3 * D_MEM, D_MSG), lambda i: (0, 0)),
            pl.BlockSpec((3 * D_MEM, D_MEM), lambda i: (0, 0)),
            pl.BlockSpec((1, 3 * D_MEM), lambda i: (0, 0)),
            pl.BlockSpec((1, 3 * D_MEM), lambda i: (0, 0)),
        ],
        out_specs=pl.BlockSpec((BR, D_MEM), lambda i: (i, 0)),
        name="tc_gru",
    )(msgs, h, W_ih, W_hh, b_ih2, b_hh2)


def _prep_body(lu_in, idx_hbm, ts_hbm, lu_out, pos2d_hbm, id2d_hbm, wcnt_hbm,
               ids_v, ts_v, cand, table, win_pos, win_id,
               pos2d, id2d, lu_v, wcnt_v):
    wid = _wid()
    lo = wid * RANGE
    iota = lax.iota(jnp.int32, 16)

    pltpu.sync_copy(idx_hbm, ids_v)
    pltpu.sync_copy(ts_hbm, ts_v)
    is_last = wid == NW - 1
    hi = jnp.where(is_last, jnp.int32(N_NODES), lo + RANGE)

    def scan_step(j, cnt):
        v_id = ids_v[pl.ds(j * 16, 16)]
        m = (v_id >= lo) & (v_id < hi)
        packed = ((j * 16 + iota) << 17) | v_id
        pref = plsc.cumsum(m.astype(jnp.int32))
        plsc.store_scatter(cand, [cnt + pref - 1], packed, mask=m)
        return cnt + pref[15]

    cnt = lax.fori_loop(0, B // 16, scan_step, jnp.int32(0))

    def init_step(j, _):
        table[pl.ds(j * 16, 16)] = jnp.full((16,), -1, jnp.int32)
        return 0

    lax.fori_loop(0, TBL_V, init_step, 0)

    lane0 = iota == 0

    def dedup_step(t, _):
        vals = cand[pl.ds(t * 16, 16)]
        for k in range(16):
            val = vals[k]
            d = (val & 0x1FFFF) - lo
            pos = val >> 17
            mk = lane0 & (t * 16 + k < cnt)
            plsc.store_scatter(
                table,
                [jnp.full((16,), d, jnp.int32)],
                jnp.full((16,), pos, jnp.int32),
                mask=mk,
            )
        return 0

    lax.fori_loop(0, (cnt + 15) // 16, dedup_step, 0)

    @pl.when(~is_last)
    def _():
        pltpu.sync_copy(lu_in.at[pl.ds(lo, RANGE)], lu_v)

    @pl.when(is_last)
    def _():
        pltpu.sync_copy(lu_in.at[pl.ds(lo, LAST_RANGE)], lu_v.at[pl.ds(0, LAST_RANGE)])

    def collect_step(j, wcnt):
        v_pos = table[pl.ds(j * 16, 16)]
        m = v_pos >= 0
        v_id = lo + j * 16 + iota
        pref = plsc.cumsum(m.astype(jnp.int32))
        offs = wcnt + pref - 1
        plsc.store_scatter(win_pos, [offs], v_pos, mask=m)
        plsc.store_scatter(win_id, [offs], v_id, mask=m)
        g = plsc.load_gather(ts_v, [jnp.maximum(v_pos, 0)])
        cur = lu_v[pl.ds(j * 16, 16)]
        lu_v[pl.ds(j * 16, 16)] = jnp.where(m, g, cur)
        return wcnt + pref[15]

    wcnt = lax.fori_loop(0, TBL_V, collect_step, jnp.int32(0))

    @pl.when(~is_last)
    def _():
        pltpu.sync_copy(lu_v, lu_out.at[pl.ds(lo, RANGE)])

    @pl.when(is_last)
    def _():
        pltpu.sync_copy(lu_v.at[pl.ds(0, LAST_RANGE)], lu_out.at[pl.ds(lo, LAST_RANGE)])

    pos0 = win_pos[pl.ds(0, 16)][0]
    id0 = win_id[pl.ds(0, 16)][0]

    def xfer_step(j, _):
        valid = (j * 16 + iota) < wcnt
        vp = jnp.where(valid, win_pos[pl.ds(j * 16, 16)], pos0)
        vi = jnp.where(valid, win_id[pl.ds(j * 16, 16)], id0)
        c = j // (CH // 16)
        k = j % (CH // 16)
        pos2d[c, pl.ds(k * 16, 16)] = vp
        id2d[c, pl.ds(k * 16, 16)] = vi
        return 0

    lax.fori_loop(0, KMAX * (CH // 16), xfer_step, 0)

    pltpu.sync_copy(pos2d, pos2d_hbm.at[pl.ds(wid * KPAD, KPAD)])
    pltpu.sync_copy(id2d, id2d_hbm.at[pl.ds(wid * KPAD, KPAD)])
    wcnt_v[...] = jnp.full((16,), 0, jnp.int32) + wcnt
    pltpu.sync_copy(wcnt_v, wcnt_hbm.at[pl.ds(wid * 16, 16)])


_prep = _mpmd._mpmd_map(
    [(_mesh, _prep_body)],
    (
        jax.ShapeDtypeStruct((N_NODES,), jnp.float32),
        jax.ShapeDtypeStruct((NW * KPAD, CH), jnp.int32),
        jax.ShapeDtypeStruct((NW * KPAD, CH), jnp.int32),
        jax.ShapeDtypeStruct((NW * 16,), jnp.int32),
    ),
    scratch_types=[
        pltpu.VMEM((B,), jnp.int32),
        pltpu.VMEM((B,), jnp.float32),
        pltpu.VMEM((B + 16,), jnp.int32),
        pltpu.VMEM((RANGE,), jnp.int32),
        pltpu.VMEM((WFLAT,), jnp.int32),
        pltpu.VMEM((WFLAT,), jnp.int32),
        pltpu.VMEM((KPAD, CH), jnp.int32),
        pltpu.VMEM((KPAD, CH), jnp.int32),
        pltpu.VMEM((RANGE,), jnp.float32),
        pltpu.VMEM((16,), jnp.int32),
    ],
    name="sc_prep",
    compiler_params=pltpu.CompilerParams(needs_layout_passes=False),
)


def _move_body(mem_in, upd_hbm, pos2d_hbm, id2d_hbm, wcnt_hbm, mem_out,
               pos2d, id2d, wcnt_v, rows0, rows1, gs0, gs1, ss0, ss1):
    wid = _wid()
    pltpu.sync_copy(pos2d_hbm.at[pl.ds(wid * KPAD, KPAD)], pos2d)
    pltpu.sync_copy(id2d_hbm.at[pl.ds(wid * KPAD, KPAD)], id2d)
    pltpu.sync_copy(wcnt_hbm.at[pl.ds(wid * 16, 16)], wcnt_v)
    wcnt = wcnt_v[pl.ds(0, 16)][0]
    nch = (wcnt + CH - 1) // CH

    @pl.when(nch > 0)
    def _():
        pltpu.async_copy(upd_hbm.at[pos2d.at[0]], rows0, gs0)

    def chunk_step(c, _):
        even = lax.rem(c, 2) == 0

        @pl.when(even)
        def _():
            pltpu.make_async_copy(upd_hbm.at[pos2d.at[c]], rows0, gs0).wait()

            @pl.when(c + 1 < nch)
            def _():
                @pl.when(c > 0)
                def _():
                    pltpu.make_async_copy(
                        rows1, mem_out.at[id2d.at[c - 1]], ss1).wait()
                pltpu.async_copy(upd_hbm.at[pos2d.at[c + 1]], rows1, gs1)
            pltpu.async_copy(rows0, mem_out.at[id2d.at[c]], ss0)

        @pl.when(~even)
        def _():
            pltpu.make_async_copy(upd_hbm.at[pos2d.at[c]], rows1, gs1).wait()

            @pl.when(c + 1 < nch)
            def _():
                pltpu.make_async_copy(
                    rows0, mem_out.at[id2d.at[c - 1]], ss0).wait()
                pltpu.async_copy(upd_hbm.at[pos2d.at[c + 1]], rows0, gs0)
            pltpu.async_copy(rows1, mem_out.at[id2d.at[c]], ss1)

        return 0

    lax.fori_loop(0, nch, chunk_step, 0)

    @pl.when(nch > 0)
    def _():
        last_even = lax.rem(nch - 1, 2) == 0

        @pl.when(last_even)
        def _():
            pltpu.make_async_copy(rows0, mem_out.at[id2d.at[nch - 1]], ss0).wait()

            @pl.when(nch > 1)
            def _():
                pltpu.make_async_copy(rows1, mem_out.at[id2d.at[nch - 2]], ss1).wait()

        @pl.when(~last_even)
        def _():
            pltpu.make_async_copy(rows1, mem_out.at[id2d.at[nch - 1]], ss1).wait()
            pltpu.make_async_copy(rows0, mem_out.at[id2d.at[nch - 2]], ss0).wait()


_move = _mpmd._mpmd_map(
    [(_mesh, _move_body)],
    jax.ShapeDtypeStruct((N_NODES, D_MEM), jnp.float32),
    input_output_aliases={0: 0},
    scratch_types=[
        pltpu.VMEM((KPAD, CH), jnp.int32),
        pltpu.VMEM((KPAD, CH), jnp.int32),
        pltpu.VMEM((16,), jnp.int32),
        pltpu.VMEM((CH, D_MEM), jnp.float32),
        pltpu.VMEM((CH, D_MEM), jnp.float32),
        pltpu.SemaphoreType.DMA,
        pltpu.SemaphoreType.DMA,
        pltpu.SemaphoreType.DMA,
        pltpu.SemaphoreType.DMA,
    ],
    name="sc_move_rows",
    compiler_params=pltpu.CompilerParams(needs_layout_passes=False),
)


def kernel(memory, last_update, unique_node_ids, unique_messages, timestamps,
           W_ih, W_hh, b_ih, b_hh):
    ids = unique_node_ids.astype(jnp.int32)
    h = _gather_call(memory, ids)
    updated = _gru_call(
        unique_messages, h, W_ih, W_hh,
        b_ih.reshape(1, 3 * D_MEM), b_hh.reshape(1, 3 * D_MEM),
    )
    new_lu, pos2d, id2d, wcnts = _prep(last_update, ids, timestamps)
    new_mem = _move(memory, updated, pos2d, id2d, wcnts)
    return (new_mem, new_lu)


# speedup vs baseline: 1.5496x; 1.0069x over previous
import functools

import jax
import jax.numpy as jnp
from jax import lax
from jax.experimental import pallas as pl
from jax.experimental.pallas import tpu as pltpu
from jax.experimental.pallas import tpu_sc as plsc
from jax._src.pallas import mpmd as _mpmd

N_NODES = 100000
D_MEM = 128
D_MSG = 256
B = 16384

NW = 32
BPW = B // NW
CH = 128
RANGE = 3136
LAST_RANGE = N_NODES - 31 * RANGE
TBL_V = RANGE // 16
KMAX = (RANGE + CH - 1) // CH
WFLAT = KMAX * CH + 16
KPAD = 32

_mesh = plsc.VectorSubcoreMesh(core_axis_name="c", subcore_axis_name="s")


def _wid():
    return lax.axis_index("s") * 2 + lax.axis_index("c")


def _gather_body(mem_hbm, idx_hbm, h_hbm, idx2, rows_v, sem):
    base = _wid() * BPW
    for k in range(BPW // CH):
        pltpu.sync_copy(idx_hbm.at[pl.ds(base + k * CH, CH)], idx2.at[k])
    cps = [
        pltpu.async_copy(mem_hbm.at[idx2.at[k]], rows_v.at[pl.ds(k * CH, CH)], sem)
        for k in range(BPW // CH)
    ]
    for cp in cps:
        cp.wait()
    pltpu.sync_copy(rows_v, h_hbm.at[pl.ds(base, BPW)])


_gather_call = pl.kernel(
    _gather_body,
    out_type=jax.ShapeDtypeStruct((B, D_MEM), jnp.float32),
    mesh=_mesh,
    scratch_types=[
        pltpu.VMEM((BPW // CH, CH), jnp.int32),
        pltpu.VMEM((BPW, D_MEM), jnp.float32),
        pltpu.SemaphoreType.DMA,
    ],
    name="sc_gather_rows",
)


def _gru_body(msgs_ref, h_ref, wih_ref, whh_ref, bih_ref, bhh_ref, upd_ref):
    h = h_ref[...]
    gi = lax.dot_general(
        msgs_ref[...], wih_ref[...], (((1,), (1,)), ((), ())),
        preferred_element_type=jnp.float32,
    ) + bih_ref[...]
    gh = lax.dot_general(
        h, whh_ref[...], (((1,), (1,)), ((), ())),
        preferred_element_type=jnp.float32,
    ) + bhh_ref[...]
    r = jax.nn.sigmoid(gi[:, :D_MEM] + gh[:, :D_MEM])
    z = jax.nn.sigmoid(gi[:, D_MEM:2 * D_MEM] + gh[:, D_MEM:2 * D_MEM])
    n = jnp.tanh(gi[:, 2 * D_MEM:] + r * gh[:, 2 * D_MEM:])
    upd_ref[...] = (1.0 - z) * n + z * h


def _gru_call(msgs, h, W_ih, W_hh, b_ih2, b_hh2):
    BR = 512
    return pl.pallas_call(
        _gru_body,
        out_shape=jax.ShapeDtypeStruct((B, D_MEM), jnp.float32),
        grid=(B // BR,),
        in_specs=[
            pl.BlockSpec((BR, D_MSG), lambda i: (i, 0)),
            pl.BlockSpec((BR, D_MEM), lambda i: (i, 0)),
            pl.BlockSpec((3 * D_MEM, D_MSG), lambda i: (0, 0)),
            pl.BlockSpec((3 * D_MEM, D_MEM), lambda i: (0, 0)),
            pl.BlockSpec((1, 3 * D_MEM), lambda i: (0, 0)),
            pl.BlockSpec((1, 3 * D_MEM), lambda i: (0, 0)),
        ],
        out_specs=pl.BlockSpec((BR, D_MEM), lambda i: (i, 0)),
        name="tc_gru",
    )(msgs, h, W_ih, W_hh, b_ih2, b_hh2)


def _prep_body(lu_in, idx_hbm, ts_hbm, lu_out, pos2d_hbm, id2d_hbm, wcnt_hbm,
               ids_v, ts_v, cand, table, win_pos, win_id,
               pos2d, id2d, lu_v, wcnt_v):
    wid = _wid()
    lo = wid * RANGE
    iota = lax.iota(jnp.int32, 16)

    pltpu.sync_copy(idx_hbm, ids_v)
    pltpu.sync_copy(ts_hbm, ts_v)
    is_last = wid == NW - 1
    hi = jnp.where(is_last, jnp.int32(N_NODES), lo + RANGE)

    def scan_step(j, cnt):
        v_id = ids_v[pl.ds(j * 16, 16)]
        m = (v_id >= lo) & (v_id < hi)
        packed = ((j * 16 + iota) << 17) | v_id
        pref = plsc.cumsum(m.astype(jnp.int32))
        plsc.store_scatter(cand, [cnt + pref - 1], packed, mask=m)
        return cnt + pref[15]

    cnt = lax.fori_loop(0, B // 16, scan_step, jnp.int32(0))

    def init_step(j, _):
        table[pl.ds(j * 16, 16)] = jnp.full((16,), -1, jnp.int32)
        return 0

    lax.fori_loop(0, TBL_V, init_step, 0)

    lane0 = iota == 0

    def dedup_step(t, _):
        vals = cand[pl.ds(t * 16, 16)]
        for k in range(16):
            val = vals[k]
            d = (val & 0x1FFFF) - lo
            pos = val >> 17
            mk = lane0 & (t * 16 + k < cnt)
            plsc.store_scatter(
                table,
                [jnp.full((16,), d, jnp.int32)],
                jnp.full((16,), pos, jnp.int32),
                mask=mk,
            )
        return 0

    lax.fori_loop(0, (cnt + 15) // 16, dedup_step, 0)

    @pl.when(~is_last)
    def _():
        pltpu.sync_copy(lu_in.at[pl.ds(lo, RANGE)], lu_v)

    @pl.when(is_last)
    def _():
        pltpu.sync_copy(lu_in.at[pl.ds(lo, LAST_RANGE)], lu_v.at[pl.ds(0, LAST_RANGE)])

    def collect_step(j, wcnt):
        v_pos = table[pl.ds(j * 16, 16)]
        m = v_pos >= 0
        v_id = lo + j * 16 + iota
        pref = plsc.cumsum(m.astype(jnp.int32))
        offs = wcnt + pref - 1
        plsc.store_scatter(win_pos, [offs], v_pos, mask=m)
        plsc.store_scatter(win_id, [offs], v_id, mask=m)
        g = plsc.load_gather(ts_v, [jnp.maximum(v_pos, 0)])
        cur = lu_v[pl.ds(j * 16, 16)]
        lu_v[pl.ds(j * 16, 16)] = jnp.where(m, g, cur)
        return wcnt + pref[15]

    wcnt = lax.fori_loop(0, TBL_V, collect_step, jnp.int32(0))

    @pl.when(~is_last)
    def _():
        pltpu.sync_copy(lu_v, lu_out.at[pl.ds(lo, RANGE)])

    @pl.when(is_last)
    def _():
        pltpu.sync_copy(lu_v.at[pl.ds(0, LAST_RANGE)], lu_out.at[pl.ds(lo, LAST_RANGE)])

    pos0 = win_pos[pl.ds(0, 16)][0]
    id0 = win_id[pl.ds(0, 16)][0]

    def xfer_step(j, _):
        valid = (j * 16 + iota) < wcnt
        vp = jnp.where(valid, win_pos[pl.ds(j * 16, 16)], pos0)
        vi = jnp.where(valid, win_id[pl.ds(j * 16, 16)], id0)
        c = j // (CH // 16)
        k = j % (CH // 16)
        pos2d[c, pl.ds(k * 16, 16)] = vp
        id2d[c, pl.ds(k * 16, 16)] = vi
        return 0

    lax.fori_loop(0, KMAX * (CH // 16), xfer_step, 0)

    pltpu.sync_copy(pos2d, pos2d_hbm.at[pl.ds(wid * KPAD, KPAD)])
    pltpu.sync_copy(id2d, id2d_hbm.at[pl.ds(wid * KPAD, KPAD)])
    wcnt_v[...] = jnp.full((16,), 0, jnp.int32) + wcnt
    pltpu.sync_copy(wcnt_v, wcnt_hbm.at[pl.ds(wid * 16, 16)])


_prep = _mpmd._mpmd_map(
    [(_mesh, _prep_body)],
    (
        jax.ShapeDtypeStruct((N_NODES,), jnp.float32),
        jax.ShapeDtypeStruct((NW * KPAD, CH), jnp.int32),
        jax.ShapeDtypeStruct((NW * KPAD, CH), jnp.int32),
        jax.ShapeDtypeStruct((NW * 16,), jnp.int32),
    ),
    scratch_types=[
        pltpu.VMEM((B,), jnp.int32),
        pltpu.VMEM((B,), jnp.float32),
        pltpu.VMEM((B + 16,), jnp.int32),
        pltpu.VMEM((RANGE,), jnp.int32),
        pltpu.VMEM((WFLAT,), jnp.int32),
        pltpu.VMEM((WFLAT,), jnp.int32),
        pltpu.VMEM((KPAD, CH), jnp.int32),
        pltpu.VMEM((KPAD, CH), jnp.int32),
        pltpu.VMEM((RANGE,), jnp.float32),
        pltpu.VMEM((16,), jnp.int32),
    ],
    name="sc_prep",
    compiler_params=pltpu.CompilerParams(needs_layout_passes=False),
)


def _move_body(mem_in, upd_hbm, pos2d_hbm, id2d_hbm, wcnt_hbm, mem_out,
               pos2d, id2d, wcnt_v, rows0, rows1, gs0, gs1, ss0, ss1):
    wid = _wid()
    pltpu.sync_copy(pos2d_hbm.at[pl.ds(wid * KPAD, KPAD)], pos2d)
    pltpu.sync_copy(id2d_hbm.at[pl.ds(wid * KPAD, KPAD)], id2d)
    pltpu.sync_copy(wcnt_hbm.at[pl.ds(wid * 16, 16)], wcnt_v)
    wcnt = wcnt_v[pl.ds(0, 16)][0]
    nch = (wcnt + CH - 1) // CH

    @pl.when(nch > 0)
    def _():
        pltpu.async_copy(upd_hbm.at[pos2d.at[0]], rows0, gs0)

    def chunk_step(c, _):
        even = lax.rem(c, 2) == 0

        @pl.when(even)
        def _():
            pltpu.make_async_copy(upd_hbm.at[pos2d.at[c]], rows0, gs0).wait()

            @pl.when(c + 1 < nch)
            def _():
                @pl.when(c > 0)
                def _():
                    pltpu.make_async_copy(
                        rows1, mem_out.at[id2d.at[c - 1]], ss1).wait()
                pltpu.async_copy(upd_hbm.at[pos2d.at[c + 1]], rows1, gs1)
            pltpu.async_copy(rows0, mem_out.at[id2d.at[c]], ss0)

        @pl.when(~even)
        def _():
            pltpu.make_async_copy(upd_hbm.at[pos2d.at[c]], rows1, gs1).wait()

            @pl.when(c + 1 < nch)
            def _():
                pltpu.make_async_copy(
                    rows0, mem_out.at[id2d.at[c - 1]], ss0).wait()
                pltpu.async_copy(upd_hbm.at[pos2d.at[c + 1]], rows0, gs0)
            pltpu.async_copy(rows1, mem_out.at[id2d.at[c]], ss1)

        return 0

    lax.fori_loop(0, nch, chunk_step, 0)

    @pl.when(nch > 0)
    def _():
        last_even = lax.rem(nch - 1, 2) == 0

        @pl.when(last_even)
        def _():
            pltpu.make_async_copy(rows0, mem_out.at[id2d.at[nch - 1]], ss0).wait()

            @pl.when(nch > 1)
            def _():
                pltpu.make_async_copy(rows1, mem_out.at[id2d.at[nch - 2]], ss1).wait()

        @pl.when(~last_even)
        def _():
            pltpu.make_async_copy(rows1, mem_out.at[id2d.at[nch - 1]], ss1).wait()
            pltpu.make_async_copy(rows0, mem_out.at[id2d.at[nch - 2]], ss0).wait()


_move = _mpmd._mpmd_map(
    [(_mesh, _move_body)],
    jax.ShapeDtypeStruct((N_NODES, D_MEM), jnp.float32),
    input_output_aliases={0: 0},
    scratch_types=[
        pltpu.VMEM((KPAD, CH), jnp.int32),
        pltpu.VMEM((KPAD, CH), jnp.int32),
        pltpu.VMEM((16,), jnp.int32),
        pltpu.VMEM((CH, D_MEM), jnp.float32),
        pltpu.VMEM((CH, D_MEM), jnp.float32),
        pltpu.SemaphoreType.DMA,
        pltpu.SemaphoreType.DMA,
        pltpu.SemaphoreType.DMA,
        pltpu.SemaphoreType.DMA,
    ],
    name="sc_move_rows",
    compiler_params=pltpu.CompilerParams(needs_layout_passes=False),
)


def kernel(memory, last_update, unique_node_ids, unique_messages, timestamps,
           W_ih, W_hh, b_ih, b_hh):
    ids = unique_node_ids.astype(jnp.int32)
    h = _gather_call(memory, ids)
    new_lu, pos2d, id2d, wcnts = _prep(last_update, ids, timestamps)
    updated = _gru_call(
        unique_messages, h, W_ih, W_hh,
        b_ih.reshape(1, 3 * D_MEM), b_hh.reshape(1, 3 * D_MEM),
    )
    new_mem = _move(memory, updated, pos2d, id2d, wcnts)
    return (new_mem, new_lu)
